# baseline (speedup 1.0000x reference)
import jax
import jax.numpy as jnp
from jax import lax
from jax.experimental import pallas as pl
from jax.experimental.pallas import tpu as pltpu

_N = 16384
_H = 256
_K = 1024
_RATE = 0.999
_BLK = 2048
_NBLK = _N // _BLK


def _rownorm(a, eps=1e-12):
    nrm = jnp.sqrt(jnp.sum(a * a, axis=1, keepdims=True))
    return a / jnp.maximum(nrm, eps)


def _body(x_ref, m_ref, score2_ref, out_ref,
          mn_ref, esumT_ref, cnt_ref, nd16_ref, mn2_ref, xn_ref):
    p = pl.program_id(0)
    i = pl.program_id(1)

    @pl.when(jnp.logical_and(p == 0, i == 0))
    def _init():
        mn_ref[...] = _rownorm(m_ref[...]).astype(jnp.float8_e4m3fn)
        esumT_ref[...] = jnp.zeros_like(esumT_ref)
        cnt_ref[...] = jnp.zeros_like(cnt_ref)

    @pl.when(p == 0)
    def _stats():
        x = x_ref[...]
        x8 = x.astype(jnp.float8_e4m3fn)
        score = lax.dot_general(x8, mn_ref[...], (((1,), (1,)), ((), ())),
                                preferred_element_type=jnp.float32)
        xn_ref[pl.ds(i * _BLK, _BLK), :] = x.astype(jnp.bfloat16)
        mx = jnp.max(score, axis=1, keepdims=True)
        oh8 = jnp.where(score == mx, 1.0, 0.0).astype(jnp.float8_e4m3fn)
        esumT_ref[...] += lax.dot_general(oh8, x8,
                                          (((0,), (0,)), ((), ())),
                                          preferred_element_type=jnp.float32)
        cnt_ref[...] += lax.dot_general(jnp.ones((1, _BLK), jnp.float8_e4m3fn),
                                        oh8, (((1,), (0,)), ((), ())),
                                        preferred_element_type=jnp.float32)

    @pl.when(jnp.logical_and(p == 1, i == 0))
    def _update():
        cntT = lax.dot_general(cnt_ref[...], jnp.ones((1, 8), jnp.float32),
                               (((0,), (0,)), ((), ())),
                               preferred_element_type=jnp.float32)
        emeanT = esumT_ref[...] / (cntT[:, 0:1] + 1e-6)
        nd = m_ref[...] * _RATE + emeanT * (1.0 - _RATE)
        mn2_ref[...] = _rownorm(nd).astype(jnp.bfloat16)
        nd16_ref[...] = nd.astype(jnp.bfloat16)

    @pl.when(p == 1)
    def _emit():
        x16 = xn_ref[pl.ds(i * _BLK, _BLK), :]
        ssq = lax.dot_general(x16 * x16, jnp.ones((_H, 8), jnp.bfloat16),
                              (((1,), (0,)), ((), ())),
                              preferred_element_type=jnp.float32)
        rinv = (1.0 / jnp.maximum(jnp.sqrt(ssq[:, 0:1]), 1e-12)
                ).astype(jnp.bfloat16)
        s2 = lax.dot_general(x16 * rinv, mn2_ref[...],
                             (((1,), (1,)), ((), ())),
                             preferred_element_type=jnp.float32)
        score2_ref[...] = s2
        e = jnp.exp(s2)
        r = 1.0 / jnp.sum(e, axis=1, keepdims=True)
        acc = lax.dot_general(e.astype(jnp.bfloat16), nd16_ref[...],
                              (((1,), (0,)), ((), ())),
                              preferred_element_type=jnp.float32)
        out_ref[...] = acc * r


def kernel(x, m):
    score2, out = pl.pallas_call(
        _body,
        grid=(2, _NBLK),
        in_specs=[pl.BlockSpec((_BLK, _H), lambda p, i: ((1 - p) * i, 0)),
                  pl.BlockSpec((_K, _H), lambda p, i: (0, 0))],
        out_specs=[pl.BlockSpec((_BLK, _K), lambda p, i: (p * i, 0)),
                   pl.BlockSpec((_BLK, _H), lambda p, i: (p * i, 0))],
        out_shape=[jax.ShapeDtypeStruct((_N, _K), jnp.float32),
                   jax.ShapeDtypeStruct((_N, _H), jnp.float32)],
        scratch_shapes=[pltpu.VMEM((_K, _H), jnp.float8_e4m3fn),
                        pltpu.VMEM((_K, _H), jnp.float32),
                        pltpu.VMEM((1, _K), jnp.float32),
                        pltpu.VMEM((_K, _H), jnp.bfloat16),
                        pltpu.VMEM((_K, _H), jnp.bfloat16),
                        pltpu.VMEM((_N, _H), jnp.bfloat16)],
    )(x, m)
    return (out, score2)


# speedup vs baseline: 1.0394x; 1.0394x over previous
import jax
import jax.numpy as jnp
from jax import lax
from jax.experimental import pallas as pl
from jax.experimental.pallas import tpu as pltpu

_N = 16384
_H = 256
_K = 1024
_RATE = 0.999
_BLK = 2048
_NBLK = _N // _BLK


def _rownorm(a, eps=1e-12):
    nrm = jnp.sqrt(jnp.sum(a * a, axis=1, keepdims=True))
    return a / jnp.maximum(nrm, eps)


def _body(x_ref, m_ref, score2_ref, out_ref,
          mn_ref, esumT_ref, cnt_ref, nd16_ref, mn2_ref, xn_ref):
    p = pl.program_id(0)
    i = pl.program_id(1)

    @pl.when(jnp.logical_and(p == 0, i == 0))
    def _init():
        mn_ref[...] = _rownorm(m_ref[...]).astype(jnp.float8_e4m3fn)
        esumT_ref[...] = jnp.zeros_like(esumT_ref)
        cnt_ref[...] = jnp.zeros_like(cnt_ref)

    @pl.when(p == 0)
    def _stats():
        x = x_ref[...]
        x8 = x.astype(jnp.float8_e4m3fn)
        score = lax.dot_general(x8, mn_ref[...], (((1,), (1,)), ((), ())),
                                preferred_element_type=jnp.float32)
        sq16 = (x * x).astype(jnp.bfloat16)
        ssq = lax.dot_general(sq16, jnp.ones((_H, 8), jnp.bfloat16),
                              (((1,), (0,)), ((), ())),
                              preferred_element_type=jnp.float32)
        rinv = 1.0 / jnp.maximum(jnp.sqrt(ssq[:, 0:1]), 1e-12)
        xn_ref[pl.ds(i * _BLK, _BLK), :] = (x * rinv).astype(jnp.bfloat16)
        mx = jnp.max(score, axis=1, keepdims=True)
        oh8 = jnp.where(score == mx, 1.0, 0.0).astype(jnp.float8_e4m3fn)
        esumT_ref[...] += lax.dot_general(oh8, x8,
                                          (((0,), (0,)), ((), ())),
                                          preferred_element_type=jnp.float32)
        cnt_ref[...] += lax.dot_general(jnp.ones((1, _BLK), jnp.float8_e4m3fn),
                                        oh8, (((1,), (0,)), ((), ())),
                                        preferred_element_type=jnp.float32)

    @pl.when(jnp.logical_and(p == 1, i == 0))
    def _update():
        cntT = lax.dot_general(cnt_ref[...], jnp.ones((1, 8), jnp.float32),
                               (((0,), (0,)), ((), ())),
                               preferred_element_type=jnp.float32)
        emeanT = esumT_ref[...] / (cntT[:, 0:1] + 1e-6)
        nd = m_ref[...] * _RATE + emeanT * (1.0 - _RATE)
        mn2_ref[...] = _rownorm(nd).astype(jnp.bfloat16)
        nd16_ref[...] = nd.astype(jnp.bfloat16)

    @pl.when(p == 1)
    def _emit():
        xn = xn_ref[pl.ds(i * _BLK, _BLK), :]
        s2 = lax.dot_general(xn, mn2_ref[...], (((1,), (1,)), ((), ())),
                             preferred_element_type=jnp.float32)
        score2_ref[...] = s2
        e = jnp.exp(s2)
        r = 1.0 / jnp.sum(e, axis=1, keepdims=True)
        acc = lax.dot_general(e.astype(jnp.bfloat16), nd16_ref[...],
                              (((1,), (0,)), ((), ())),
                              preferred_element_type=jnp.float32)
        out_ref[...] = acc * r


def kernel(x, m):
    score2, out = pl.pallas_call(
        _body,
        grid=(2, _NBLK),
        in_specs=[pl.BlockSpec((_BLK, _H), lambda p, i: ((1 - p) * i, 0)),
                  pl.BlockSpec((_K, _H), lambda p, i: (0, 0))],
        out_specs=[pl.BlockSpec((_BLK, _K), lambda p, i: (p * i, 0)),
                   pl.BlockSpec((_BLK, _H), lambda p, i: (p * i, 0))],
        out_shape=[jax.ShapeDtypeStruct((_N, _K), jnp.float32),
                   jax.ShapeDtypeStruct((_N, _H), jnp.float32)],
        scratch_shapes=[pltpu.VMEM((_K, _H), jnp.float8_e4m3fn),
                        pltpu.VMEM((_K, _H), jnp.float32),
                        pltpu.VMEM((1, _K), jnp.float32),
                        pltpu.VMEM((_K, _H), jnp.bfloat16),
                        pltpu.VMEM((_K, _H), jnp.bfloat16),
                        pltpu.VMEM((_N, _H), jnp.bfloat16)],
    )(x, m)
    return (out, score2)
